# Initial kernel scaffold; baseline (speedup 1.0000x reference)
import jax
import jax.numpy as jnp
from jax.experimental import pallas as pl


def kernel(x, edge_index, W1, b1, W2, b2, W3, b3, Wfc, bfc):
    raise NotImplementedError("write your pallas kernel here")


# speedup vs baseline: 69.8298x; 69.8298x over previous
import functools

import jax
import jax.numpy as jnp
from jax import lax
from jax.experimental import pallas as pl
from jax.experimental.pallas import tpu as pltpu
from jax.experimental.pallas import tpu_sc as plsc

N_NODES = 10000
N_EDGES = 320000
NP = 10240
NC = 2
NS = 16
NW = NC * NS
RPT = NP // NS
EPW = N_EDGES // NW
EPT = N_EDGES // NS

_mesh = plsc.VectorSubcoreMesh(
    core_axis_name="c", subcore_axis_name="s", num_cores=NC, num_subcores=NS
)
_sc_params = pltpu.CompilerParams(needs_layout_passes=False)


def _rsqrt16(d):
    i = lax.bitcast_convert_type(d, jnp.int32)
    i = jnp.int32(0x5F3759DF) - lax.shift_right_logical(i, 1)
    y = lax.bitcast_convert_type(i, jnp.float32)
    for _ in range(3):
        y = y * (1.5 - 0.5 * d * y * y)
    return y


def _tanh16(a):
    return 1.0 - 2.0 / (jnp.exp(2.0 * a) + 1.0)


def _deg_body(dst_hbm, s_out, dstv, onesv, zbuf, dbuf, sbuf, accd):
    c = lax.axis_index("c")
    sid = lax.axis_index("s")

    @pl.when(c == 0)
    def _():
        r0 = sid * RPT

        def fill(i, _):
            zbuf[pl.ds(i * 16, 16)] = jnp.zeros((16,), jnp.float32)
            return _

        lax.fori_loop(0, RPT // 16, fill, None)
        pltpu.sync_copy(zbuf, accd.at[pl.ds(r0, RPT)])

        def ones(i, _):
            onesv[pl.ds(i * 16, 16)] = jnp.full((16,), 1.0, jnp.float32)
            return _

        lax.fori_loop(0, EPT // 16, ones, None)
        pltpu.sync_copy(dst_hbm.at[pl.ds(sid * EPT, EPT)], dstv)
        plsc.subcore_barrier()
        pltpu.sync_copy(onesv, accd.at[dstv], add=True)
        plsc.subcore_barrier()
        pltpu.sync_copy(accd.at[pl.ds(r0, RPT)], dbuf)

        def conv(i, _):
            sl = pl.ds(i * 16, 16)
            sbuf[sl] = _rsqrt16(dbuf[sl] + 1.0)
            return _

        lax.fori_loop(0, RPT // 16, conv, None)
        pltpu.sync_copy(sbuf, s_out.at[pl.ds(r0, RPT)])


_deg_kernel = functools.partial(
    pl.kernel,
    out_type=jax.ShapeDtypeStruct((NP,), jnp.float32),
    mesh=_mesh,
    scratch_types=[
        pltpu.VMEM((EPT,), jnp.int32),
        pltpu.VMEM((EPT,), jnp.float32),
        pltpu.VMEM((RPT,), jnp.float32),
        pltpu.VMEM((RPT,), jnp.float32),
        pltpu.VMEM((RPT,), jnp.float32),
        pltpu.VMEM_SHARED((NP,), jnp.float32),
    ],
    compiler_params=_sc_params,
)(_deg_body)


def _make_layer(first, wp, wo):
    def body(*refs):
        if first:
            (s_hbm, prev, src_hbm, dst_hbm, u_out, part_out,
             sv, pv, ubuf, zbuf, srcv, dstv, uS, *rest) = refs
            wv = None
            agv = None
        else:
            (s_hbm, prev, partp, params, src_hbm, dst_hbm, u_out, part_out,
             sv, pv, agv, wv, ubuf, zbuf, srcv, dstv, uS, *rest) = refs
        ufull = rest[:wo]
        msg = rest[wo:2 * wo]
        accS = rest[2 * wo:]

        c = lax.axis_index("c")
        sid = lax.axis_index("s")
        wid = c * NS + sid
        r0 = sid * RPT
        slc = pl.ds(r0, RPT)

        pltpu.sync_copy(s_hbm.at[slc], sv)
        for f in range(wp):
            pltpu.sync_copy(prev.at[f, slc], pv.at[f])
        if not first:
            for cc in range(NC):
                for f in range(wp):
                    pltpu.sync_copy(partp.at[cc, f, slc], agv.at[cc, f])
            pltpu.sync_copy(params, wv)

        if not first:
            w_lo = wv[pl.ds(0, 16)]
            w_hi = wv[pl.ds(16, 16)]

            def wval(k):
                return w_lo[k] if k < 16 else w_hi[k - 16]

        def grp(i, _):
            sl = pl.ds(i * 16, 16)
            s16 = sv[sl]
            if first:
                for j in range(wo):
                    ubuf[j, sl] = s16 * pv[j, sl]
            else:
                h = []
                for f in range(wp):
                    a = (agv[0, f, sl] + agv[1, f, sl] + pv[f, sl]) * s16 + wval(f)
                    h.append(_tanh16(a))
                for j in range(wo):
                    z = h[0] * wval(wp + j)
                    for f in range(1, wp):
                        z = z + h[f] * wval(wp + f * wo + j)
                    ubuf[j, sl] = s16 * z
            zbuf[sl] = jnp.zeros((16,), jnp.float32)
            return _

        lax.fori_loop(0, RPT // 16, grp, None)

        for f in range(wo):
            pltpu.sync_copy(ubuf.at[f], uS.at[f, slc])
            pltpu.sync_copy(zbuf, accS[f].at[slc])

        @pl.when(c == 0)
        def _():
            for f in range(wo):
                pltpu.sync_copy(ubuf.at[f], u_out.at[f, slc])

        plsc.subcore_barrier()

        for f in range(wo):
            pltpu.sync_copy(uS.at[f], ufull[f])
        e0 = wid * EPW
        pltpu.sync_copy(src_hbm.at[pl.ds(e0, EPW)], srcv)
        pltpu.sync_copy(dst_hbm.at[pl.ds(e0, EPW)], dstv)

        def ggrp(g, _):
            sl = pl.ds(g * 16, 16)
            s16 = srcv[sl]
            for f in range(wo):
                msg[f][sl] = plsc.load_gather(ufull[f], [s16])
            return _

        lax.fori_loop(0, EPW // 16, ggrp, None)

        for f in range(wo):
            pltpu.sync_copy(msg[f], accS[f].at[dstv], add=True)
        plsc.subcore_barrier()

        for f in range(wo):
            pltpu.sync_copy(accS[f].at[slc], part_out.at[c, f, slc])

    scratch = [
        pltpu.VMEM((RPT,), jnp.float32),
        pltpu.VMEM((wp, RPT), jnp.float32),
    ]
    if not first:
        scratch += [
            pltpu.VMEM((NC, wp, RPT), jnp.float32),
            pltpu.VMEM((32,), jnp.float32),
        ]
    scratch += [
        pltpu.VMEM((wo, RPT), jnp.float32),
        pltpu.VMEM((RPT,), jnp.float32),
        pltpu.VMEM((EPW,), jnp.int32),
        pltpu.VMEM((EPW,), jnp.int32),
        pltpu.VMEM_SHARED((wo, NP), jnp.float32),
    ]
    scratch += [pltpu.VMEM((NP,), jnp.float32) for _ in range(wo)]
    scratch += [pltpu.VMEM((EPW,), jnp.float32) for _ in range(wo)]
    scratch += [pltpu.VMEM_SHARED((NP,), jnp.float32) for _ in range(wo)]

    return functools.partial(
        pl.kernel,
        out_type=(
            jax.ShapeDtypeStruct((wo, NP), jnp.float32),
            jax.ShapeDtypeStruct((NC, wo, NP), jnp.float32),
        ),
        mesh=_mesh,
        scratch_types=scratch,
        compiler_params=_sc_params,
    )(body)


_layer1 = _make_layer(True, 4, 4)
_layer2 = _make_layer(False, 4, 4)
_layer3 = _make_layer(False, 4, 2)


def _mm_body(x_ref, w_ref, o_ref):
    o_ref[...] = jnp.dot(x_ref[...], w_ref[...],
                         preferred_element_type=jnp.float32)


def _xw1(x_pad, w1):
    blk = 1024
    return pl.pallas_call(
        _mm_body,
        grid=(NP // blk,),
        in_specs=[
            pl.BlockSpec((blk, 128), lambda i: (i, 0)),
            pl.BlockSpec((128, 4), lambda i: (0, 0)),
        ],
        out_specs=pl.BlockSpec((blk, 4), lambda i: (i, 0)),
        out_shape=jax.ShapeDtypeStruct((NP, 4), jnp.float32),
    )(x_pad, w1)


def _head_body(s_ref, u_ref, p_ref, b3_ref, wfc_ref, bfc_ref, h_ref, o_ref):
    pre = (p_ref[0] + p_ref[1] + u_ref[...]) * s_ref[...] + b3_ref[...]
    h = jnp.tanh(pre)
    h_ref[...] = h
    o_ref[...] = jnp.dot(wfc_ref[...], h,
                         preferred_element_type=jnp.float32) + bfc_ref[...]


def _head(s_row, u3, part3, b3_col, wfc_t, bfc_col):
    return pl.pallas_call(
        _head_body,
        out_shape=(
            jax.ShapeDtypeStruct((2, NP), jnp.float32),
            jax.ShapeDtypeStruct((8, NP), jnp.float32),
        ),
    )(s_row, u3, part3, b3_col, wfc_t, bfc_col)


@jax.jit
def kernel(x, edge_index, W1, b1, W2, b2, W3, b3, Wfc, bfc):
    src = jnp.asarray(edge_index[0], jnp.int32)
    dst = jnp.asarray(edge_index[1], jnp.int32)

    x_pad = jnp.pad(x, ((0, NP - N_NODES), (0, 0)))
    z1t = _xw1(x_pad, W1).T

    s = _deg_kernel(dst)

    u1, part1 = _layer1(s, z1t, src, dst)

    p12 = jnp.concatenate([b1, W2.reshape(-1)])
    params12 = jnp.pad(p12, (0, 32 - p12.shape[0]))
    u2, part2 = _layer2(s, u1, part1, params12, src, dst)

    p23 = jnp.concatenate([b2, W3.reshape(-1)])
    params23 = jnp.pad(p23, (0, 32 - p23.shape[0]))
    u3, part3 = _layer3(s, u2, part2, params23, src, dst)

    h4t, out_t = _head(
        s.reshape(1, NP), u3, part3,
        b3.reshape(2, 1), Wfc.T, bfc.reshape(8, 1),
    )

    return out_t.T[:N_NODES], h4t.T[:N_NODES]
